# baseline (speedup 1.0000x reference)
import functools

import numpy as np
import jax
import jax.numpy as jnp
from jax import lax
from jax.experimental import pallas as pl
from jax.experimental.pallas import tpu as pltpu
from jax.experimental.pallas import tpu_sc as plsc

_OMEGA_SCALE = 10000


def _pos_table(seqlen, dims):
    positions = np.arange(seqlen)[:, np.newaxis]
    omega = 1 / np.power(_OMEGA_SCALE, 2 * (np.arange(dims, dtype=int) // 2) / np.float64(dims))
    emb = positions * omega
    emb[:, 0::2] = np.sin(emb[:, 0::2])
    emb[:, 1::2] = np.cos(emb[:, 1::2])
    return emb.astype(np.float32)


@functools.lru_cache(maxsize=None)
def _make_embed(V, D, N, S):
    info = plsc.get_sparse_core_info()
    NC, NS, L = info.num_cores, info.num_subcores, info.num_lanes
    NW = NC * NS
    n_per_w = N // NW
    W = 100
    C = 4 * S
    G = C // W
    n_chunks = n_per_w // C
    VN = D // L
    assert N % NW == 0 and n_per_w % C == 0 and C % W == 0 and C % S == 0
    assert N % W == 0 and D % L == 0 and G % 8 == 0

    mesh = plsc.VectorSubcoreMesh(core_axis_name="c", subcore_axis_name="s")

    @functools.partial(
        pl.kernel, mesh=mesh,
        compiler_params=pltpu.CompilerParams(use_tc_tiling_on_sc=False),
        out_type=jax.ShapeDtypeStruct((N // S, S, D), jnp.float32),
        scratch_types=[
            pltpu.VMEM((G, W), jnp.int32),
            pltpu.VMEM((C // S, S, D), jnp.float32),
            pltpu.VMEM((S, D), jnp.float32),
            pltpu.SemaphoreType.DMA,
        ],
    )
    def k(table_hbm, idx_hbm, pos_hbm, out_hbm, idx_v, rows_v, pos_v, sem):
        wid = lax.axis_index("s") * NC + lax.axis_index("c")
        base = wid * n_per_w
        pltpu.sync_copy(pos_hbm, pos_v)

        def chunk_body(c, carry):
            cbase = pl.multiple_of(base + c * C, C)
            pltpu.sync_copy(idx_hbm.at[pl.ds(pl.multiple_of(cbase // W, 8), G)], idx_v)
            copies = [
                pltpu.async_copy(table_hbm.at[idx_v.at[j]],
                                 rows_v.at[(j * W) // S, pl.ds((j * W) % S, W)],
                                 sem)
                for j in range(G)
            ]
            for cp in copies:
                cp.wait()

            def add_body(s, acc):
                for v in range(VN):
                    p = pos_v[s, pl.ds(v * L, L)]
                    for q in range(C // S):
                        plsc.addupdate(rows_v.at[q, s, pl.ds(v * L, L)], p)
                return acc

            lax.fori_loop(0, S, add_body, 0)
            pltpu.sync_copy(rows_v, out_hbm.at[pl.ds(cbase // S, C // S)])
            return carry

        lax.fori_loop(0, n_chunks, chunk_body, 0)

    return k


def kernel(inputs, table):
    B, S = inputs.shape
    V, D = table.shape
    N = B * S
    flat = inputs.reshape(N).astype(jnp.int32)
    idx2d = flat.reshape(N // 100, 100)
    pos = jnp.asarray(_pos_table(S, D))
    out = _make_embed(V, D, N, S)(table, idx2d, pos)
    return out.reshape(B, S, D)


# speedup vs baseline: 1.0086x; 1.0086x over previous
import functools

import numpy as np
import jax
import jax.numpy as jnp
from jax import lax
from jax.experimental import pallas as pl
from jax.experimental.pallas import tpu as pltpu
from jax.experimental.pallas import tpu_sc as plsc

_OMEGA_SCALE = 10000


def _pos_table(seqlen, dims):
    positions = np.arange(seqlen)[:, np.newaxis]
    omega = 1 / np.power(_OMEGA_SCALE, 2 * (np.arange(dims, dtype=int) // 2) / np.float64(dims))
    emb = positions * omega
    emb[:, 0::2] = np.sin(emb[:, 0::2])
    emb[:, 1::2] = np.cos(emb[:, 1::2])
    return emb.astype(np.float32)


@functools.lru_cache(maxsize=None)
def _make_embed(V, D, N, S):
    info = plsc.get_sparse_core_info()
    NC, NS, L = info.num_cores, info.num_subcores, info.num_lanes
    NW = NC * NS
    n_per_w = N // NW
    W = 100
    C = 8 * S
    G = C // W
    n_chunks = n_per_w // C
    VN = D // L
    assert N % NW == 0 and n_per_w % C == 0 and C % W == 0 and C % S == 0
    assert N % W == 0 and D % L == 0 and G % 8 == 0

    mesh = plsc.VectorSubcoreMesh(core_axis_name="c", subcore_axis_name="s")

    @functools.partial(
        pl.kernel, mesh=mesh,
        compiler_params=pltpu.CompilerParams(use_tc_tiling_on_sc=False),
        out_type=jax.ShapeDtypeStruct((N // S, S, D), jnp.float32),
        scratch_types=[
            pltpu.VMEM((G, W), jnp.int32),
            pltpu.VMEM((C // S, S, D), jnp.float32),
            pltpu.VMEM((S, D), jnp.float32),
            pltpu.SemaphoreType.DMA,
        ],
    )
    def k(table_hbm, idx_hbm, pos_hbm, out_hbm, idx_v, rows_v, pos_v, sem):
        wid = lax.axis_index("s") * NC + lax.axis_index("c")
        base = wid * n_per_w
        pltpu.sync_copy(pos_hbm, pos_v)

        def chunk_body(c, carry):
            cbase = pl.multiple_of(base + c * C, C)
            pltpu.sync_copy(idx_hbm.at[pl.ds(pl.multiple_of(cbase // W, 8), G)], idx_v)
            copies = [
                pltpu.async_copy(table_hbm.at[idx_v.at[j]],
                                 rows_v.at[(j * W) // S, pl.ds((j * W) % S, W)],
                                 sem)
                for j in range(G)
            ]
            for cp in copies:
                cp.wait()

            def add_body(s, acc):
                for v in range(VN):
                    p = pos_v[s, pl.ds(v * L, L)]
                    for q in range(C // S):
                        plsc.addupdate(rows_v.at[q, s, pl.ds(v * L, L)], p)
                return acc

            lax.fori_loop(0, S, add_body, 0)
            pltpu.sync_copy(rows_v, out_hbm.at[pl.ds(cbase // S, C // S)])
            return carry

        lax.fori_loop(0, n_chunks, chunk_body, 0)

    return k


def kernel(inputs, table):
    B, S = inputs.shape
    V, D = table.shape
    N = B * S
    flat = inputs.reshape(N).astype(jnp.int32)
    idx2d = flat.reshape(N // 100, 100)
    pos = jnp.asarray(_pos_table(S, D))
    out = _make_embed(V, D, N, S)(table, idx2d, pos)
    return out.reshape(B, S, D)
